# baseline (speedup 1.0000x reference)
import jax
import jax.numpy as jnp
from jax.experimental import pallas as pl
from jax.experimental.pallas import tpu as pltpu

_NR = 16
_UNROLL = 4


def _detection_kernel(rois_ref, ids_ref, gt_ref, out_ref):
    r = rois_ref[0]
    y1 = r[0]
    x1 = r[1]
    y2 = r[2]
    x2 = r[3]
    a1 = (y2 - y1) * (x2 - x1)
    G = gt_ref.shape[1]
    init = jnp.full_like(a1, -1.0)

    def gbody(g, carry):
        nc, cb = carry
        gy1 = gt_ref[0, g, 0]
        gx1 = gt_ref[0, g, 1]
        gy2 = gt_ref[0, g, 2]
        gx2 = gt_ref[0, g, 3]
        gid = ids_ref[0, g, 0]
        valid = ((jnp.abs(gy1) > 0) | (jnp.abs(gx1) > 0) |
                 (jnp.abs(gy2) > 0) | (jnp.abs(gx2) > 0))
        one = jnp.float32(1.0)
        zero = jnp.float32(0.0)
        neg1 = jnp.float32(-1.0)
        is_nc = valid & (gid > 0)
        is_c = valid & (gid < 0)
        w_nc = jnp.where(is_nc, one, zero)
        b_nc = jnp.where(is_nc, zero, neg1)
        w_c = jnp.where(is_c, one, zero)
        b_c = jnp.where(is_c, zero, neg1)

        a2 = (gy2 - gy1) * (gx2 - gx1)
        iy1 = jnp.maximum(y1, gy1)
        ix1 = jnp.maximum(x1, gx1)
        iy2 = jnp.minimum(y2, gy2)
        ix2 = jnp.minimum(x2, gx2)
        inter = jnp.maximum(iy2 - iy1, 0.0) * jnp.maximum(ix2 - ix1, 0.0)
        union = a1 + a2 - inter
        iou = inter / jnp.maximum(union, 1e-8)
        nc = jnp.maximum(nc, iou * w_nc + b_nc)
        cb = jnp.maximum(cb, iou * w_c + b_c)
        return nc, cb

    nc_max, c_max = jax.lax.fori_loop(0, G, gbody, (init, init),
                                      unroll=_UNROLL)

    roi_valid = ((jnp.abs(y1) > 0) | (jnp.abs(x1) > 0) |
                 (jnp.abs(y2) > 0) | (jnp.abs(x2) > 0))
    neg_one = jnp.float32(-1.0)
    nc_max = jnp.where(roi_valid, nc_max, neg_one)
    c_max = jnp.where(roi_valid, c_max, neg_one)
    pos = ((nc_max >= 0.5) & roi_valid).astype(jnp.float32)
    neg = ((nc_max < 0.5) & (c_max < 0.001) & roi_valid).astype(jnp.float32)
    out_ref[0, 0] = nc_max
    out_ref[0, 1] = c_max
    out_ref[0, 2] = pos
    out_ref[0, 3] = neg


def kernel(rois, gt_ids, gt_boxes):
    B, N, _ = rois.shape
    G = gt_boxes.shape[1]
    NL = N // _NR
    rb = _NR // 8
    rois_t = jnp.transpose(rois, (0, 2, 1)).reshape(B, 4, _NR, NL)
    out = pl.pallas_call(
        _detection_kernel,
        grid=(B, rb),
        in_specs=[
            pl.BlockSpec((1, 4, 8, NL), lambda b, r: (b, 0, r, 0)),
            pl.BlockSpec((1, G, 1), lambda b, r: (b, 0, 0),
                         memory_space=pltpu.SMEM),
            pl.BlockSpec((1, G, 4), lambda b, r: (b, 0, 0),
                         memory_space=pltpu.SMEM),
        ],
        out_specs=pl.BlockSpec((1, 4, 8, NL), lambda b, r: (b, 0, r, 0)),
        out_shape=jax.ShapeDtypeStruct((B, 4, _NR, NL), jnp.float32),
        compiler_params=pltpu.CompilerParams(
            dimension_semantics=("parallel", "parallel"),
        ),
    )(rois_t, gt_ids.reshape(B, G, 1), gt_boxes)
    return out.reshape(B, 4, N)


# speedup vs baseline: 1.0699x; 1.0699x over previous
import jax
import jax.numpy as jnp
from jax.experimental import pallas as pl
from jax.experimental.pallas import tpu as pltpu

_NR = 8
_UNROLL = 2


def _detection_kernel(rois_ref, ids_ref, gt_ref, out_ref):
    r = rois_ref[0]
    y1 = r[0]
    x1 = r[1]
    y2 = r[2]
    x2 = r[3]
    a1 = (y2 - y1) * (x2 - x1)
    G = gt_ref.shape[1]
    init = jnp.full_like(a1, -1.0)

    def gbody(g, carry):
        nc, cb = carry
        gy1 = gt_ref[0, g, 0]
        gx1 = gt_ref[0, g, 1]
        gy2 = gt_ref[0, g, 2]
        gx2 = gt_ref[0, g, 3]
        gid = ids_ref[0, g, 0]
        valid = ((jnp.abs(gy1) > 0) | (jnp.abs(gx1) > 0) |
                 (jnp.abs(gy2) > 0) | (jnp.abs(gx2) > 0))
        neg1 = jnp.float32(-1.0)
        is_nc = valid & (gid > 0)
        is_c = valid & (gid < 0)

        a2 = (gy2 - gy1) * (gx2 - gx1)
        iy1 = jnp.maximum(y1, gy1)
        ix1 = jnp.maximum(x1, gx1)
        iy2 = jnp.minimum(y2, gy2)
        ix2 = jnp.minimum(x2, gx2)
        inter = jnp.maximum(iy2 - iy1, 0.0) * jnp.maximum(ix2 - ix1, 0.0)
        union = a1 + a2 - inter
        iou = inter / jnp.maximum(union, 1e-8)
        nc = jnp.maximum(nc, jnp.where(is_nc, iou, neg1))
        cb = jnp.maximum(cb, jnp.where(is_c, iou, neg1))
        return nc, cb

    nc_max, c_max = jax.lax.fori_loop(0, G, gbody, (init, init),
                                      unroll=_UNROLL)

    roi_valid = ((jnp.abs(y1) > 0) | (jnp.abs(x1) > 0) |
                 (jnp.abs(y2) > 0) | (jnp.abs(x2) > 0))
    neg_one = jnp.float32(-1.0)
    nc_max = jnp.where(roi_valid, nc_max, neg_one)
    c_max = jnp.where(roi_valid, c_max, neg_one)
    pos = ((nc_max >= 0.5) & roi_valid).astype(jnp.float32)
    neg = ((nc_max < 0.5) & (c_max < 0.001) & roi_valid).astype(jnp.float32)
    out_ref[0, 0] = nc_max
    out_ref[0, 1] = c_max
    out_ref[0, 2] = pos
    out_ref[0, 3] = neg


def kernel(rois, gt_ids, gt_boxes):
    B, N, _ = rois.shape
    G = gt_boxes.shape[1]
    NL = N // _NR
    rb = _NR // 8
    rois_t = jnp.transpose(rois, (0, 2, 1)).reshape(B, 4, _NR, NL)
    out = pl.pallas_call(
        _detection_kernel,
        grid=(B, rb),
        in_specs=[
            pl.BlockSpec((1, 4, 8, NL), lambda b, r: (b, 0, r, 0)),
            pl.BlockSpec((1, G, 1), lambda b, r: (b, 0, 0),
                         memory_space=pltpu.SMEM),
            pl.BlockSpec((1, G, 4), lambda b, r: (b, 0, 0),
                         memory_space=pltpu.SMEM),
        ],
        out_specs=pl.BlockSpec((1, 4, 8, NL), lambda b, r: (b, 0, r, 0)),
        out_shape=jax.ShapeDtypeStruct((B, 4, _NR, NL), jnp.float32),
        compiler_params=pltpu.CompilerParams(
            dimension_semantics=("parallel", "parallel"),
        ),
    )(rois_t, gt_ids.reshape(B, G, 1), gt_boxes)
    return out.reshape(B, 4, N)
